# baseline (speedup 1.0000x reference)
import functools

import jax
import jax.numpy as jnp
from jax.experimental import pallas as pl
from jax.experimental.pallas import tpu as pltpu

BEAM = 4
VOCAB = 100000
KV_ELEMS = 8 * 2048 * 64
COPY_GRID = 32
COPY_CHUNK = KV_ELEMS // COPY_GRID

NEG_BIG = -1e30


def _copy_body(*refs):
    n = len(refs) // 2
    ins, outs = refs[:n], refs[n:]
    for i_ref, o_ref in zip(ins, outs):
        o_ref[...] = jnp.broadcast_to(i_ref[...], (BEAM, COPY_CHUNK))


def _beam_tile(kvs):
    flat = [kv.reshape(1, KV_ELEMS) for kv in kvs]
    grid_spec = pl.GridSpec(
        grid=(COPY_GRID,),
        in_specs=[pl.BlockSpec((1, COPY_CHUNK), lambda g: (0, g))] * len(kvs),
        out_specs=[pl.BlockSpec((BEAM, COPY_CHUNK), lambda g: (0, g))] * len(kvs),
    )
    outs = pl.pallas_call(
        _copy_body,
        grid_spec=grid_spec,
        out_shape=[jax.ShapeDtypeStruct((BEAM, KV_ELEMS), jnp.float32)] * len(kvs),
    )(*flat)
    return [o.reshape(BEAM, 8, 2048, 64) for o in outs]


def _topk_body(logits_ref, rp_ref, pen_ref,
               idx_ref, prob_ref, rp_out_ref):
    x = logits_ref[...]
    m = jnp.max(x)
    s = jnp.sum(jnp.exp(x - m))
    lse = m + jnp.log(s)

    col = jax.lax.broadcasted_iota(jnp.int32, (1, VOCAB), 1)
    v = x
    idxs = []
    vals = []
    for _ in range(BEAM):
        mv = jnp.max(v)
        ii = jnp.min(jnp.where(v == mv, col, VOCAB))
        idxs.append(ii)
        vals.append(mv - lse)
        v = jnp.where(col == ii, NEG_BIG, v)

    row = jax.lax.broadcasted_iota(jnp.int32, (BEAM, 1), 0)
    iv = idxs[0]
    pv = vals[0]
    for t in range(1, BEAM):
        iv = jnp.where(row == t, idxs[t], iv)
        pv = jnp.where(row == t, vals[t], pv)
    idx_ref[...] = iv
    prob_ref[...] = pv

    pen = pen_ref[0]
    colb = jax.lax.broadcasted_iota(jnp.int32, (BEAM, VOCAB), 1)
    hit = (colb == idxs[0]) | (colb == idxs[1]) | (colb == idxs[2]) | (colb == idxs[3])
    rp_out_ref[...] = rp_ref[...] * jnp.where(hit, pen, jnp.float32(1.0))


def _topk_penalty(logits, repeat_penality, penality_value):
    return pl.pallas_call(
        _topk_body,
        in_specs=[
            pl.BlockSpec(memory_space=None),
            pl.BlockSpec(memory_space=None),
            pl.BlockSpec(memory_space=pltpu.SMEM),
        ],
        out_shape=[
            jax.ShapeDtypeStruct((BEAM, 1), jnp.int32),
            jax.ShapeDtypeStruct((BEAM, 1), jnp.float32),
            jax.ShapeDtypeStruct((BEAM, VOCAB), jnp.float32),
        ],
    )(logits, repeat_penality, penality_value)


def kernel(kv_0, kv_1, kv_2, kv_3, kv_4, kv_5, kv_6, kv_7, kv_8, kv_9,
           kv_10, kv_11, kv_12, kv_13, kv_14, kv_15,
           logits, save_id, repeat_penality, penality_value, beam_size):
    kvs = [kv_0, kv_1, kv_2, kv_3, kv_4, kv_5, kv_6, kv_7,
           kv_8, kv_9, kv_10, kv_11, kv_12, kv_13, kv_14, kv_15]
    saved = _beam_tile(kvs)
    top_idx, top_prob, rp_out = _topk_penalty(
        logits, repeat_penality, penality_value)
    beam = save_id.shape[0]
    save_id_out = jnp.concatenate([save_id, top_idx], axis=-1)
    batch_indices = jnp.arange(beam, dtype=jnp.int32) + (
        jnp.asarray(beam_size, dtype=jnp.int32) - beam)
    max_logits_idx = top_idx[0]
    return (*saved, top_idx, save_id_out, rp_out, top_prob,
            batch_indices, max_logits_idx)


# speedup vs baseline: 2.0038x; 2.0038x over previous
import functools

import jax
import jax.numpy as jnp
from jax.experimental import pallas as pl
from jax.experimental.pallas import tpu as pltpu

BEAM = 4
VOCAB = 100000
KV_ELEMS = 8 * 2048 * 64
COPY_GRID = 32
COPY_CHUNK = KV_ELEMS // COPY_GRID

NEG_BIG = -1e30


KV_ROWS = KV_ELEMS // 128
ROW_CHUNK = KV_ROWS // COPY_GRID


def _copy_body(*refs):
    n = len(refs) // 2
    ins, outs = refs[:n], refs[n:]
    for i_ref, o_ref in zip(ins, outs):
        o_ref[...] = jnp.broadcast_to(i_ref[...][None], (BEAM, ROW_CHUNK, 128))


def _beam_tile(kvs):
    flat = [kv.reshape(KV_ROWS, 128) for kv in kvs]
    grid_spec = pl.GridSpec(
        grid=(COPY_GRID,),
        in_specs=[pl.BlockSpec((ROW_CHUNK, 128), lambda g: (g, 0))] * len(kvs),
        out_specs=[pl.BlockSpec((BEAM, ROW_CHUNK, 128), lambda g: (0, g, 0))] * len(kvs),
    )
    outs = pl.pallas_call(
        _copy_body,
        grid_spec=grid_spec,
        out_shape=[jax.ShapeDtypeStruct((BEAM, KV_ROWS, 128), jnp.float32)] * len(kvs),
    )(*flat)
    return [o.reshape(BEAM, 8, 2048, 64) for o in outs]


def _topk_body(logits_ref, rp_ref, pen_ref,
               idx_ref, prob_ref, rp_out_ref):
    x = logits_ref[...]
    m = jnp.max(x)
    s = jnp.sum(jnp.exp(x - m))
    lse = m + jnp.log(s)

    col = jax.lax.broadcasted_iota(jnp.int32, (1, VOCAB), 1)
    v = x
    idxs = []
    vals = []
    for _ in range(BEAM):
        mv = jnp.max(v)
        ii = jnp.min(jnp.where(v == mv, col, VOCAB))
        idxs.append(ii)
        vals.append(mv - lse)
        v = jnp.where(col == ii, NEG_BIG, v)

    row = jax.lax.broadcasted_iota(jnp.int32, (BEAM, 1), 0)
    iv = idxs[0]
    pv = vals[0]
    for t in range(1, BEAM):
        iv = jnp.where(row == t, idxs[t], iv)
        pv = jnp.where(row == t, vals[t], pv)
    idx_ref[...] = iv
    prob_ref[...] = pv

    pen = pen_ref[0]
    colb = jax.lax.broadcasted_iota(jnp.int32, (BEAM, VOCAB), 1)
    hit = (colb == idxs[0]) | (colb == idxs[1]) | (colb == idxs[2]) | (colb == idxs[3])
    rp_out_ref[...] = rp_ref[...] * jnp.where(hit, pen, jnp.float32(1.0))


def _topk_penalty(logits, repeat_penality, penality_value):
    return pl.pallas_call(
        _topk_body,
        in_specs=[
            pl.BlockSpec(memory_space=None),
            pl.BlockSpec(memory_space=None),
            pl.BlockSpec(memory_space=pltpu.SMEM),
        ],
        out_shape=[
            jax.ShapeDtypeStruct((BEAM, 1), jnp.int32),
            jax.ShapeDtypeStruct((BEAM, 1), jnp.float32),
            jax.ShapeDtypeStruct((BEAM, VOCAB), jnp.float32),
        ],
    )(logits, repeat_penality, penality_value)


def kernel(kv_0, kv_1, kv_2, kv_3, kv_4, kv_5, kv_6, kv_7, kv_8, kv_9,
           kv_10, kv_11, kv_12, kv_13, kv_14, kv_15,
           logits, save_id, repeat_penality, penality_value, beam_size):
    kvs = [kv_0, kv_1, kv_2, kv_3, kv_4, kv_5, kv_6, kv_7,
           kv_8, kv_9, kv_10, kv_11, kv_12, kv_13, kv_14, kv_15]
    saved = _beam_tile(kvs)
    top_idx, top_prob, rp_out = _topk_penalty(
        logits, repeat_penality, penality_value)
    beam = save_id.shape[0]
    save_id_out = jnp.concatenate([save_id, top_idx], axis=-1)
    batch_indices = jnp.arange(beam, dtype=jnp.int32) + (
        jnp.asarray(beam_size, dtype=jnp.int32) - beam)
    max_logits_idx = top_idx[0]
    return (*saved, top_idx, save_id_out, rp_out, top_prob,
            batch_indices, max_logits_idx)
